# Initial kernel scaffold; baseline (speedup 1.0000x reference)
import jax
import jax.numpy as jnp
from jax.experimental import pallas as pl


def kernel(x, neighbour_distance, enc_W, enc_b, dec_W, bw_W, bw_b, neighbour_id, clustering_labels):
    raise NotImplementedError("write your pallas kernel here")


# speedup vs baseline: 1.3601x; 1.3601x over previous
import functools

import jax
import jax.numpy as jnp
from jax import lax
from jax.experimental import pallas as pl
from jax.experimental.pallas import tpu as pltpu
from jax.experimental.pallas import tpu_sc as plsc

_N = 10000
_n = 64
_m = 16
_K = 16
_B = 4
_MU = 600.0
_BJ = _B * _n

_IB = 400
_NB = _N // _IB

_NW = 32
_CH = 128
_NCH = 40


def _prep_body(x_ref, encwt_ref, encb_ref, bwwt_ref, bwb_ref,
               enc_out, araw_out):
    enc = lax.dot_general(x_ref[...], encwt_ref[...],
                          (((1,), (0,)), ((), ())),
                          preferred_element_type=jnp.float32)
    enc = enc + encb_ref[...]
    logits = lax.dot_general(enc, bwwt_ref[...],
                             (((1,), (0,)), ((), ())),
                             preferred_element_type=jnp.float32)
    logits = logits + bwb_ref[...]
    t = (60.0 / _MU) * (1.0 + jnp.exp(-logits))
    enc_out[...] = enc
    araw_out[...] = t * t


def _prep(x, enc_wt, enc_b, bw_wt, bw_b):
    return pl.pallas_call(
        _prep_body,
        out_shape=(
            jax.ShapeDtypeStruct((_B, _n), jnp.float32),
            jax.ShapeDtypeStruct((_B, _n * _m), jnp.float32),
        ),
    )(x, enc_wt, enc_b, bw_wt, bw_b)


def _window_block(lab_ref, nd_ref, acm_ref):
    lab = lab_ref[0]
    ioc = lax.broadcasted_iota(jnp.int32, (_m, _IB), 0)
    oht = jnp.where(lab == ioc, 1.0, 0.0)
    a_blk = lax.dot_general(oht, acm_ref[...],
                            (((0,), (0,)), ((), ())),
                            preferred_element_type=jnp.float32)
    ndb = nd_ref[...]
    return a_blk, ndb * ndb


def _s_body(lab_ref, nd_ref, acm_ref, encr_ref, e_out, s_acc):
    t = pl.program_id(0)
    a_blk, d2 = _window_block(lab_ref, nd_ref, acm_ref)
    rows = []
    for k in range(_K):
        wk = jnp.maximum(1.0 - a_blk * d2[:, k:k + 1], 0.0)
        rows.append(jnp.sum(wk, axis=0, keepdims=True))
    s_new = jnp.concatenate(rows, axis=0)

    @pl.when(t == 0)
    def _():
        s_acc[...] = s_new

    @pl.when(t != 0)
    def _():
        s_acc[...] = s_acc[...] + s_new

    @pl.when(t == _NB - 1)
    def _():
        e_out[...] = encr_ref[...] / s_acc[...]


def _s_stage(labels3, nd, a_cm, enc_r):
    return pl.pallas_call(
        _s_body,
        grid=(_NB,),
        in_specs=[
            pl.BlockSpec((1, 1, _IB), lambda t: (t, 0, 0)),
            pl.BlockSpec((_IB, _K), lambda t: (t, 0)),
            pl.BlockSpec((_m, _BJ), lambda t: (0, 0)),
            pl.BlockSpec((1, _BJ), lambda t: (0, 0)),
        ],
        out_specs=pl.BlockSpec((_K, _BJ), lambda t: (0, 0)),
        out_shape=jax.ShapeDtypeStruct((_K, _BJ), jnp.float32),
        scratch_shapes=[pltpu.VMEM((_K, _BJ), jnp.float32)],
    )(labels3, nd, a_cm, enc_r)


def _sc_gather_body(decwt_hbm, nid_hbm, out_hbm, idx_v, rows_v, sem):
    wid = lax.axis_index("s") * 2 + lax.axis_index("c")
    pltpu.sync_copy(nid_hbm.at[wid], idx_v)

    def chunk(j, carry):
        pltpu.async_copy(decwt_hbm.at[idx_v.at[j]], rows_v, sem).wait()
        pltpu.sync_copy(rows_v, out_hbm.at[wid, j])
        return carry

    lax.fori_loop(0, _NCH, chunk, 0)


def _sc_gather(dec_wt, nid3):
    fn = functools.partial(
        pl.kernel,
        out_type=jax.ShapeDtypeStruct((_NW, _NCH, _CH, _n), jnp.float32),
        mesh=plsc.VectorSubcoreMesh(core_axis_name="c", subcore_axis_name="s"),
        scratch_types=[
            pltpu.VMEM((_NCH, _CH), jnp.int32),
            pltpu.VMEM((_CH, _n), jnp.float32),
            pltpu.SemaphoreType.DMA,
        ],
        compiler_params=pltpu.CompilerParams(use_tc_tiling_on_sc=False),
    )(_sc_gather_body)
    return fn(dec_wt, nid3)


def _main_body(g_ref, lab_ref, nd_ref, acm_ref, e_ref, out_ref):
    a_blk, d2 = _window_block(lab_ref, nd_ref, acm_ref)
    e_full = e_ref[...]
    g_full = g_ref[...]
    p = None
    for k in range(_K):
        wk = jnp.maximum(1.0 - a_blk * d2[:, k:k + 1], 0.0)
        wk = wk * e_full[k:k + 1, :]
        gk = g_full[:, k * _n:(k + 1) * _n]
        gk4 = jnp.concatenate([gk, gk, gk, gk], axis=1)
        term = wk * gk4
        p = term if p is None else p + term
    rr = lax.broadcasted_iota(jnp.int32, (_BJ, _B), 0) // _n
    cc = lax.broadcasted_iota(jnp.int32, (_BJ, _B), 1)
    bd = jnp.where(rr == cc, 1.0, 0.0)
    out_ref[...] = lax.dot_general(p, bd, (((1,), (0,)), ((), ())),
                                   preferred_element_type=jnp.float32)


def _main(g, labels3, nd, a_cm, e_tab):
    return pl.pallas_call(
        _main_body,
        grid=(_NB,),
        in_specs=[
            pl.BlockSpec((_IB, _K * _n), lambda t: (t, 0)),
            pl.BlockSpec((1, 1, _IB), lambda t: (t, 0, 0)),
            pl.BlockSpec((_IB, _K), lambda t: (t, 0)),
            pl.BlockSpec((_m, _BJ), lambda t: (0, 0)),
            pl.BlockSpec((_K, _BJ), lambda t: (0, 0)),
        ],
        out_specs=pl.BlockSpec((_IB, _B), lambda t: (t, 0)),
        out_shape=jax.ShapeDtypeStruct((_N, _B), jnp.float32),
    )(g, labels3, nd, a_cm, e_tab)


def kernel(x, neighbour_distance, enc_W, enc_b, dec_W, bw_W, bw_b,
           neighbour_id, clustering_labels):
    enc_wt = enc_W.T
    bw_wt = bw_W.T
    dec_wt = dec_W.T

    encoded, araw = _prep(x, enc_wt, enc_b.reshape(1, _n),
                          bw_wt, bw_b.reshape(1, _n * _m))
    a_cm = araw.reshape(_B, _n, _m).transpose(2, 0, 1).reshape(_m, _BJ)
    enc_r = encoded.reshape(1, _BJ)
    labels3 = clustering_labels.reshape(_NB, 1, _IB)

    e_tab = _s_stage(labels3, neighbour_distance, a_cm, enc_r)

    nid_flat = neighbour_id.reshape(-1)
    pad = _NW * _NCH * _CH - nid_flat.shape[0]
    nid3 = jnp.concatenate(
        [nid_flat, jnp.zeros((pad,), jnp.int32)]).reshape(_NW, _NCH, _CH)
    g4 = _sc_gather(dec_wt, nid3)
    g = g4.reshape(-1, _n)[: _N * _K].reshape(_N, _K * _n)

    out_t = _main(g, labels3, neighbour_distance, a_cm, e_tab)
    return out_t.T


# speedup vs baseline: 3.2025x; 2.3546x over previous
import functools

import jax
import jax.numpy as jnp
from jax import lax
from jax.experimental import pallas as pl
from jax.experimental.pallas import tpu as pltpu
from jax.experimental.pallas import tpu_sc as plsc

_N = 10000
_n = 64
_m = 16
_K = 16
_B = 4
_MU = 600.0
_BJ = _B * _n

_IB = 400
_NB = _N // _IB

_NW = 32
_RPW = _N * _K // _NW
_CH = 128
_NF = _RPW // _CH
_TAIL = _RPW - _NF * _CH
_NBUF = 4


def _prep_body(x_ref, encwt_ref, encb_ref, bwwt_ref, bwb_ref,
               enc_out, araw_out):
    enc = lax.dot_general(x_ref[...], encwt_ref[...],
                          (((1,), (0,)), ((), ())),
                          preferred_element_type=jnp.float32)
    enc = enc + encb_ref[...]
    logits = lax.dot_general(enc, bwwt_ref[...],
                             (((1,), (0,)), ((), ())),
                             preferred_element_type=jnp.float32)
    logits = logits + bwb_ref[...]
    t = (60.0 / _MU) * (1.0 + jnp.exp(-logits))
    enc_out[...] = enc
    araw_out[...] = t * t


def _prep(x, enc_wt, enc_b, bw_wt, bw_b):
    return pl.pallas_call(
        _prep_body,
        out_shape=(
            jax.ShapeDtypeStruct((_B, _n), jnp.float32),
            jax.ShapeDtypeStruct((_B, _n * _m), jnp.float32),
        ),
    )(x, enc_wt, enc_b, bw_wt, bw_b)


def _window_block(lab_ref, nd_ref, acm_ref):
    lab = lab_ref[0]
    ioc = lax.broadcasted_iota(jnp.int32, (_m, _IB), 0)
    oht = jnp.where(lab == ioc, 1.0, 0.0)
    a_blk = lax.dot_general(oht, acm_ref[...],
                            (((0,), (0,)), ((), ())),
                            preferred_element_type=jnp.float32)
    ndb = nd_ref[...]
    return a_blk, ndb * ndb


def _s_body(lab_ref, nd_ref, acm_ref, encr_ref, e_out, s_acc):
    t = pl.program_id(0)
    a_blk, d2 = _window_block(lab_ref, nd_ref, acm_ref)
    rows = []
    for k in range(_K):
        wk = jnp.maximum(1.0 - a_blk * d2[:, k:k + 1], 0.0)
        rows.append(jnp.sum(wk, axis=0, keepdims=True))
    s_new = jnp.concatenate(rows, axis=0)

    @pl.when(t == 0)
    def _():
        s_acc[...] = s_new

    @pl.when(t != 0)
    def _():
        s_acc[...] = s_acc[...] + s_new

    @pl.when(t == _NB - 1)
    def _():
        e_out[...] = encr_ref[...] / s_acc[...]


def _s_stage(labels3, nd, a_cm, enc_r):
    return pl.pallas_call(
        _s_body,
        grid=(_NB,),
        in_specs=[
            pl.BlockSpec((1, 1, _IB), lambda t: (t, 0, 0)),
            pl.BlockSpec((_IB, _K), lambda t: (t, 0)),
            pl.BlockSpec((_m, _BJ), lambda t: (0, 0)),
            pl.BlockSpec((1, _BJ), lambda t: (0, 0)),
        ],
        out_specs=pl.BlockSpec((_K, _BJ), lambda t: (0, 0)),
        out_shape=jax.ShapeDtypeStruct((_K, _BJ), jnp.float32),
        scratch_shapes=[pltpu.VMEM((_K, _BJ), jnp.float32)],
    )(labels3, nd, a_cm, enc_r)


def _sc_gather_body(decwt_hbm, nid_hbm, out_hbm, idx_v, *scr):
    bufs = scr[:_NBUF]
    tailbuf = scr[_NBUF]
    gsems = scr[_NBUF + 1:2 * _NBUF + 1]
    ssems = scr[2 * _NBUF + 1:3 * _NBUF + 1]
    wid = lax.axis_index("s") * 2 + lax.axis_index("c")
    pltpu.sync_copy(nid_hbm.at[wid], idx_v)

    def gather(j):
        b = j % _NBUF
        return pltpu.async_copy(
            decwt_hbm.at[idx_v.at[pl.ds(j * _CH, _CH)]], bufs[b], gsems[b])

    def store(j):
        b = j % _NBUF
        return pltpu.async_copy(
            bufs[b], out_hbm.at[wid, pl.ds(j * _CH, _CH)], ssems[b])

    gops = [None] * _NF
    sops = [None] * _NF
    for j in range(_NF):
        if j >= _NBUF:
            sops[j - _NBUF].wait()
        gops[j] = gather(j)
        if j >= 2:
            gops[j - 2].wait()
            sops[j - 2] = store(j - 2)
    for j in range(max(_NF - 2, 0), _NF):
        gops[j].wait()
        sops[j] = store(j)
    tg = pltpu.async_copy(
        decwt_hbm.at[idx_v.at[pl.ds(_NF * _CH, _TAIL)]], tailbuf, gsems[0])
    tg.wait()
    pltpu.sync_copy(tailbuf, out_hbm.at[wid, pl.ds(_NF * _CH, _TAIL)])
    for j in range(max(_NF - _NBUF, 0), _NF):
        sops[j].wait()


def _sc_gather(dec_wt, nid2):
    scratch = ([pltpu.VMEM((_RPW,), jnp.int32)]
               + [pltpu.VMEM((_CH, _n), jnp.float32) for _ in range(_NBUF)]
               + [pltpu.VMEM((_TAIL, _n), jnp.float32)]
               + [pltpu.SemaphoreType.DMA for _ in range(2 * _NBUF)])
    fn = functools.partial(
        pl.kernel,
        out_type=jax.ShapeDtypeStruct((_NW, _RPW, _n), jnp.float32),
        mesh=plsc.VectorSubcoreMesh(core_axis_name="c", subcore_axis_name="s"),
        scratch_types=scratch,
        compiler_params=pltpu.CompilerParams(use_tc_tiling_on_sc=False),
    )(_sc_gather_body)
    return fn(dec_wt, nid2)


def _main_body(g_ref, lab_ref, nd_ref, acm_ref, e_ref, out_ref):
    a_blk, d2 = _window_block(lab_ref, nd_ref, acm_ref)
    e_full = e_ref[...]
    g_full = g_ref[...]
    p = None
    for k in range(_K):
        wk = jnp.maximum(1.0 - a_blk * d2[:, k:k + 1], 0.0)
        wk = wk * e_full[k:k + 1, :]
        gk = g_full[:, k * _n:(k + 1) * _n]
        gk4 = jnp.concatenate([gk, gk, gk, gk], axis=1)
        term = wk * gk4
        p = term if p is None else p + term
    rr = lax.broadcasted_iota(jnp.int32, (_BJ, _B), 0) // _n
    cc = lax.broadcasted_iota(jnp.int32, (_BJ, _B), 1)
    bd = jnp.where(rr == cc, 1.0, 0.0)
    out_ref[...] = lax.dot_general(p, bd, (((1,), (0,)), ((), ())),
                                   preferred_element_type=jnp.float32)


def _main(g, labels3, nd, a_cm, e_tab):
    return pl.pallas_call(
        _main_body,
        grid=(_NB,),
        in_specs=[
            pl.BlockSpec((_IB, _K * _n), lambda t: (t, 0)),
            pl.BlockSpec((1, 1, _IB), lambda t: (t, 0, 0)),
            pl.BlockSpec((_IB, _K), lambda t: (t, 0)),
            pl.BlockSpec((_m, _BJ), lambda t: (0, 0)),
            pl.BlockSpec((_K, _BJ), lambda t: (0, 0)),
        ],
        out_specs=pl.BlockSpec((_IB, _B), lambda t: (t, 0)),
        out_shape=jax.ShapeDtypeStruct((_N, _B), jnp.float32),
    )(g, labels3, nd, a_cm, e_tab)


def kernel(x, neighbour_distance, enc_W, enc_b, dec_W, bw_W, bw_b,
           neighbour_id, clustering_labels):
    enc_wt = enc_W.T
    bw_wt = bw_W.T
    dec_wt = dec_W.T

    encoded, araw = _prep(x, enc_wt, enc_b.reshape(1, _n),
                          bw_wt, bw_b.reshape(1, _n * _m))
    a_cm = araw.reshape(_B, _n, _m).transpose(2, 0, 1).reshape(_m, _BJ)
    enc_r = encoded.reshape(1, _BJ)
    labels3 = clustering_labels.reshape(_NB, 1, _IB)

    e_tab = _s_stage(labels3, neighbour_distance, a_cm, enc_r)

    nid2 = neighbour_id.reshape(_NW, _RPW)
    g3 = _sc_gather(dec_wt, nid2)
    g = g3.reshape(_N, _K * _n)

    out_t = _main(g, labels3, neighbour_distance, a_cm, e_tab)
    return out_t.T


# speedup vs baseline: 3.9212x; 1.2244x over previous
import functools

import jax
import jax.numpy as jnp
from jax import lax
from jax.experimental import pallas as pl
from jax.experimental.pallas import tpu as pltpu
from jax.experimental.pallas import tpu_sc as plsc

_N = 10000
_n = 64
_m = 16
_K = 16
_B = 4
_MU = 600.0
_BJ = _B * _n

_IB = 1000
_NB = _N // _IB

_NW = 32
_RPW = _N * _K // _NW
_CH = 128
_NF = _RPW // _CH
_TAIL = _RPW - _NF * _CH
_NBUF = 4


def _prep_body(x_ref, encwt_ref, encb_ref, bwwt_ref, bwb_ref,
               enc_out, araw_out):
    enc = lax.dot_general(x_ref[...], encwt_ref[...],
                          (((1,), (0,)), ((), ())),
                          preferred_element_type=jnp.float32)
    enc = enc + encb_ref[...]
    logits = lax.dot_general(enc, bwwt_ref[...],
                             (((1,), (0,)), ((), ())),
                             preferred_element_type=jnp.float32)
    logits = logits + bwb_ref[...]
    t = (60.0 / _MU) * (1.0 + jnp.exp(-logits))
    enc_out[...] = enc
    araw_out[...] = t * t


def _prep(x, enc_wt, enc_b, bw_wt, bw_b):
    return pl.pallas_call(
        _prep_body,
        out_shape=(
            jax.ShapeDtypeStruct((_B, _n), jnp.float32),
            jax.ShapeDtypeStruct((_B, _n * _m), jnp.float32),
        ),
    )(x, enc_wt, enc_b, bw_wt, bw_b)


def _window_block(lab_ref, nd_ref, acm_ref):
    lab = lab_ref[0]
    ioc = lax.broadcasted_iota(jnp.int32, (_m, _IB), 0)
    oht = jnp.where(lab == ioc, 1.0, 0.0)
    a_blk = lax.dot_general(oht, acm_ref[...],
                            (((0,), (0,)), ((), ())),
                            preferred_element_type=jnp.float32)
    ndb = nd_ref[...]
    return a_blk, ndb * ndb


def _s_body(lab_ref, nd_ref, acm_ref, encr_ref, e_out, s_acc):
    t = pl.program_id(0)
    a_blk, d2 = _window_block(lab_ref, nd_ref, acm_ref)
    ones_row = jnp.ones((1, _IB), jnp.float32)
    rows = []
    for k in range(_K):
        wk = jnp.maximum(1.0 - a_blk * d2[:, k:k + 1], 0.0)
        rows.append(lax.dot_general(ones_row, wk, (((1,), (0,)), ((), ())),
                                    preferred_element_type=jnp.float32))
    s_new = jnp.concatenate(rows, axis=0)

    @pl.when(t == 0)
    def _():
        s_acc[...] = s_new

    @pl.when(t != 0)
    def _():
        s_acc[...] = s_acc[...] + s_new

    @pl.when(t == _NB - 1)
    def _():
        e_out[...] = encr_ref[...] / s_acc[...]


def _s_stage(labels3, nd, a_cm, enc_r):
    return pl.pallas_call(
        _s_body,
        grid=(_NB,),
        in_specs=[
            pl.BlockSpec((1, 1, _IB), lambda t: (t, 0, 0)),
            pl.BlockSpec((_IB, _K), lambda t: (t, 0)),
            pl.BlockSpec((_m, _BJ), lambda t: (0, 0)),
            pl.BlockSpec((1, _BJ), lambda t: (0, 0)),
        ],
        out_specs=pl.BlockSpec((_K, _BJ), lambda t: (0, 0)),
        out_shape=jax.ShapeDtypeStruct((_K, _BJ), jnp.float32),
        scratch_shapes=[pltpu.VMEM((_K, _BJ), jnp.float32)],
    )(labels3, nd, a_cm, enc_r)


def _sc_gather_body(decwt_hbm, nid_hbm, out_hbm, idx_v, *scr):
    bufs = scr[:_NBUF]
    tailbuf = scr[_NBUF]
    gsems = scr[_NBUF + 1:2 * _NBUF + 1]
    ssems = scr[2 * _NBUF + 1:3 * _NBUF + 1]
    wid = lax.axis_index("s") * 2 + lax.axis_index("c")
    pltpu.sync_copy(nid_hbm.at[wid], idx_v)

    def gather(j):
        b = j % _NBUF
        return pltpu.async_copy(
            decwt_hbm.at[idx_v.at[pl.ds(j * _CH, _CH)]], bufs[b], gsems[b])

    def store(j):
        b = j % _NBUF
        return pltpu.async_copy(
            bufs[b], out_hbm.at[wid, pl.ds(j * _CH, _CH)], ssems[b])

    gops = [None] * _NF
    sops = [None] * _NF
    for j in range(_NF):
        if j >= _NBUF:
            sops[j - _NBUF].wait()
        gops[j] = gather(j)
        if j >= 2:
            gops[j - 2].wait()
            sops[j - 2] = store(j - 2)
    for j in range(max(_NF - 2, 0), _NF):
        gops[j].wait()
        sops[j] = store(j)
    tg = pltpu.async_copy(
        decwt_hbm.at[idx_v.at[pl.ds(_NF * _CH, _TAIL)]], tailbuf, gsems[0])
    tg.wait()
    pltpu.sync_copy(tailbuf, out_hbm.at[wid, pl.ds(_NF * _CH, _TAIL)])
    for j in range(max(_NF - _NBUF, 0), _NF):
        sops[j].wait()


def _sc_gather(dec_wt, nid2):
    scratch = ([pltpu.VMEM((_RPW,), jnp.int32)]
               + [pltpu.VMEM((_CH, _n), jnp.float32) for _ in range(_NBUF)]
               + [pltpu.VMEM((_TAIL, _n), jnp.float32)]
               + [pltpu.SemaphoreType.DMA for _ in range(2 * _NBUF)])
    fn = functools.partial(
        pl.kernel,
        out_type=jax.ShapeDtypeStruct((_NW, _RPW, _n), jnp.float32),
        mesh=plsc.VectorSubcoreMesh(core_axis_name="c", subcore_axis_name="s"),
        scratch_types=scratch,
        compiler_params=pltpu.CompilerParams(use_tc_tiling_on_sc=False),
    )(_sc_gather_body)
    return fn(dec_wt, nid2)


def _main_body(g_ref, lab_ref, nd_ref, acm_ref, e_ref, out_ref):
    a_blk, d2 = _window_block(lab_ref, nd_ref, acm_ref)
    e_full = e_ref[...]
    p = None
    for k in range(_K):
        wk = jnp.maximum(1.0 - a_blk * d2[:, k:k + 1], 0.0)
        wk = wk * e_full[k:k + 1, :]
        gq = g_ref[:, k // 2, :]
        gk = gq[:, (k % 2) * _n:(k % 2 + 1) * _n]
        gk4 = jnp.concatenate([gk, gk, gk, gk], axis=1)
        term = wk * gk4
        p = term if p is None else p + term
    rr = lax.broadcasted_iota(jnp.int32, (_BJ, _B), 0) // _n
    cc = lax.broadcasted_iota(jnp.int32, (_BJ, _B), 1)
    bd = jnp.where(rr == cc, 1.0, 0.0)
    out_ref[...] = lax.dot_general(p, bd, (((1,), (0,)), ((), ())),
                                   preferred_element_type=jnp.float32)


def _main(g, labels3, nd, a_cm, e_tab):
    return pl.pallas_call(
        _main_body,
        grid=(_NB,),
        in_specs=[
            pl.BlockSpec((_IB, _K * _n // 128, 128), lambda t: (t, 0, 0)),
            pl.BlockSpec((1, 1, _IB), lambda t: (t, 0, 0)),
            pl.BlockSpec((_IB, _K), lambda t: (t, 0)),
            pl.BlockSpec((_m, _BJ), lambda t: (0, 0)),
            pl.BlockSpec((_K, _BJ), lambda t: (0, 0)),
        ],
        out_specs=pl.BlockSpec((_IB, _B), lambda t: (t, 0)),
        out_shape=jax.ShapeDtypeStruct((_N, _B), jnp.float32),
    )(g, labels3, nd, a_cm, e_tab)


def kernel(x, neighbour_distance, enc_W, enc_b, dec_W, bw_W, bw_b,
           neighbour_id, clustering_labels):
    enc_wt = enc_W.T
    bw_wt = bw_W.T
    dec_wt = dec_W.T

    encoded, araw = _prep(x, enc_wt, enc_b.reshape(1, _n),
                          bw_wt, bw_b.reshape(1, _n * _m))
    a_cm = araw.reshape(_B, _n, _m).transpose(2, 0, 1).reshape(_m, _BJ)
    enc_r = encoded.reshape(1, _BJ)
    labels3 = clustering_labels.reshape(_NB, 1, _IB)

    e_tab = _s_stage(labels3, neighbour_distance, a_cm, enc_r)

    nid2 = neighbour_id.reshape(_NW, _RPW)
    g3 = _sc_gather(dec_wt, nid2)
    g = g3.reshape(_N, _K * _n // 128, 128)

    out_t = _main(g, labels3, neighbour_distance, a_cm, e_tab)
    return out_t.T


# speedup vs baseline: 4.4137x; 1.1256x over previous
import functools

import jax
import jax.numpy as jnp
from jax import lax
from jax.experimental import pallas as pl
from jax.experimental.pallas import tpu as pltpu
from jax.experimental.pallas import tpu_sc as plsc

_N = 10000
_n = 64
_m = 16
_K = 16
_B = 4
_MU = 600.0
_BJ = _B * _n

_IB = 2000
_NB = _N // _IB

_NW = 32
_RPW = _N * _K // _NW
_CH = 128
_NF = _RPW // _CH
_TAIL = _RPW - _NF * _CH
_NBUF = 4


def _prep_body(x_ref, encw_ref, bww_ref, encb_ref, bwb_ref,
               enct_out, araw_out):
    xt = jnp.transpose(x_ref[...])
    enct = lax.dot_general(encw_ref[...], xt,
                           (((1,), (0,)), ((), ())),
                           preferred_element_type=jnp.float32)
    enct = enct + encb_ref[...]
    logt = lax.dot_general(bww_ref[...], enct,
                           (((1,), (0,)), ((), ())),
                           preferred_element_type=jnp.float32)
    logt = logt + bwb_ref[...]
    tt = (60.0 / _MU) * (1.0 + jnp.exp(-logt))
    enct_out[...] = enct
    araw_out[...] = tt * tt


def _prep(x, enc_w, bw_w, encb_col, bwb_col):
    return pl.pallas_call(
        _prep_body,
        out_shape=(
            jax.ShapeDtypeStruct((_n, _B), jnp.float32),
            jax.ShapeDtypeStruct((_n * _m, _B), jnp.float32),
        ),
    )(x, enc_w, bw_w, encb_col, bwb_col)


def _window_block(lab_ref, nd_ref, acm_ref):
    lab = lab_ref[0]
    ioc = lax.broadcasted_iota(jnp.int32, (_m, _IB), 0)
    oht = jnp.where(lab == ioc, 1.0, 0.0)
    a_blk = lax.dot_general(oht, acm_ref[...],
                            (((0,), (0,)), ((), ())),
                            preferred_element_type=jnp.float32)
    ndb = nd_ref[...]
    return a_blk, ndb * ndb


def _s_body(lab_ref, nd_ref, acm_ref, encr_ref, e_out, s_acc):
    t = pl.program_id(0)
    a_blk, d2 = _window_block(lab_ref, nd_ref, acm_ref)
    ones_row = jnp.ones((1, _IB), jnp.float32)
    rows = []
    for k in range(_K):
        wk = jnp.maximum(1.0 - a_blk * d2[:, k:k + 1], 0.0)
        rows.append(lax.dot_general(ones_row, wk, (((1,), (0,)), ((), ())),
                                    preferred_element_type=jnp.float32))
    s_new = jnp.concatenate(rows, axis=0)

    @pl.when(t == 0)
    def _():
        s_acc[...] = s_new

    @pl.when(t != 0)
    def _():
        s_acc[...] = s_acc[...] + s_new

    @pl.when(t == _NB - 1)
    def _():
        e_out[...] = encr_ref[...] / s_acc[...]


def _s_stage(labels3, nd, a_cm, enc_r):
    return pl.pallas_call(
        _s_body,
        grid=(_NB,),
        in_specs=[
            pl.BlockSpec((1, 1, _IB), lambda t: (t, 0, 0)),
            pl.BlockSpec((_IB, _K), lambda t: (t, 0)),
            pl.BlockSpec((_m, _BJ), lambda t: (0, 0)),
            pl.BlockSpec((1, _BJ), lambda t: (0, 0)),
        ],
        out_specs=pl.BlockSpec((_K, _BJ), lambda t: (0, 0)),
        out_shape=jax.ShapeDtypeStruct((_K, _BJ), jnp.float32),
        scratch_shapes=[pltpu.VMEM((_K, _BJ), jnp.float32)],
    )(labels3, nd, a_cm, enc_r)


def _sc_gather_body(decwt_hbm, nid_hbm, out_hbm, idx_v, *scr):
    bufs = scr[:_NBUF]
    tailbuf = scr[_NBUF]
    gsems = scr[_NBUF + 1:2 * _NBUF + 1]
    ssems = scr[2 * _NBUF + 1:3 * _NBUF + 1]
    wid = lax.axis_index("s") * 2 + lax.axis_index("c")
    pltpu.sync_copy(nid_hbm.at[wid], idx_v)

    def gather(j):
        b = j % _NBUF
        return pltpu.async_copy(
            decwt_hbm.at[idx_v.at[pl.ds(j * _CH, _CH)]], bufs[b], gsems[b])

    def store(j):
        b = j % _NBUF
        return pltpu.async_copy(
            bufs[b], out_hbm.at[wid, pl.ds(j * _CH, _CH)], ssems[b])

    gops = [None] * _NF
    sops = [None] * _NF
    for j in range(_NF):
        if j >= _NBUF:
            sops[j - _NBUF].wait()
        gops[j] = gather(j)
        if j >= 2:
            gops[j - 2].wait()
            sops[j - 2] = store(j - 2)
    for j in range(max(_NF - 2, 0), _NF):
        gops[j].wait()
        sops[j] = store(j)
    tg = pltpu.async_copy(
        decwt_hbm.at[idx_v.at[pl.ds(_NF * _CH, _TAIL)]], tailbuf, gsems[0])
    tg.wait()
    pltpu.sync_copy(tailbuf, out_hbm.at[wid, pl.ds(_NF * _CH, _TAIL)])
    for j in range(max(_NF - _NBUF, 0), _NF):
        sops[j].wait()


def _sc_gather(dec_wt, nid2):
    scratch = ([pltpu.VMEM((_RPW,), jnp.int32)]
               + [pltpu.VMEM((_CH, _n), jnp.float32) for _ in range(_NBUF)]
               + [pltpu.VMEM((_TAIL, _n), jnp.float32)]
               + [pltpu.SemaphoreType.DMA for _ in range(2 * _NBUF)])
    fn = functools.partial(
        pl.kernel,
        out_type=jax.ShapeDtypeStruct((_NW, _RPW, _n), jnp.float32),
        mesh=plsc.VectorSubcoreMesh(core_axis_name="c", subcore_axis_name="s"),
        scratch_types=scratch,
        compiler_params=pltpu.CompilerParams(use_tc_tiling_on_sc=False),
    )(_sc_gather_body)
    return fn(dec_wt, nid2)


def _main_body(g_hbm, lab_ref, nd_ref, acm_ref, e_ref, out_ref, *scr):
    t = pl.program_id(0)
    lab = lab_ref[0]
    ioc = lax.broadcasted_iota(jnp.int32, (_m, _IB), 0)
    oht = jnp.where(lab == ioc, 1.0, 0.0)
    acm = acm_ref[...]
    acm_dup = jnp.concatenate(
        [jnp.concatenate([acm[:, b * _n:(b + 1) * _n]] * 2, axis=1)
         for b in range(_B)], axis=1)
    a2 = lax.dot_general(oht, acm_dup, (((0,), (0,)), ((), ())),
                         preferred_element_type=jnp.float32)
    a_b = [a2[:, b * 128:(b + 1) * 128] for b in range(_B)]
    nd = nd_ref[...]
    ndsq = nd * nd
    e_full = e_ref[...]
    r_k = lax.broadcasted_iota(jnp.int32, (_K, 128), 0)
    c_half = lax.broadcasted_iota(jnp.int32, (_K, 128), 1) // _n

    nq = _K // 2
    gbufs, sems = scr[:4], scr[4:]
    copies = [None] * nq

    def start(qq):
        copies[qq] = pltpu.make_async_copy(
            g_hbm.at[pl.ds(t * _IB, _IB), qq, :], gbufs[qq % 4], sems[qq % 4])
        copies[qq].start()

    for qq in range(3):
        start(qq)
    p = [None] * _B
    for q in range(nq):
        if q + 3 < nq:
            start(q + 3)
        copies[q].wait()
        gq = gbufs[q % 4][...]
        sel = jnp.where(r_k == 2 * q + c_half, 1.0, 0.0)
        d2q = lax.dot_general(ndsq, sel, (((1,), (0,)), ((), ())),
                              preferred_element_type=jnp.float32)
        for b in range(_B):
            erow = e_full[q:q + 1, b * 128:(b + 1) * 128]
            w = jnp.maximum(1.0 - a_b[b] * d2q, 0.0)
            term = w * gq * erow
            p[b] = term if p[b] is None else p[b] + term
    pcat = jnp.concatenate(p, axis=1)
    rr = lax.broadcasted_iota(jnp.int32, (4 * 128, _B), 0) // 128
    cc = lax.broadcasted_iota(jnp.int32, (4 * 128, _B), 1)
    bd = jnp.where(rr == cc, 1.0, 0.0)
    res = lax.dot_general(pcat, bd, (((1,), (0,)), ((), ())),
                          preferred_element_type=jnp.float32)
    out_ref[0] = jnp.transpose(res)


def _main(g, labels3, nd, a_cm, e_tab):
    return pl.pallas_call(
        _main_body,
        grid=(_NB,),
        in_specs=[
            pl.BlockSpec(memory_space=pl.ANY),
            pl.BlockSpec((1, 1, _IB), lambda t: (t, 0, 0)),
            pl.BlockSpec((_IB, _K), lambda t: (t, 0)),
            pl.BlockSpec((_m, _BJ), lambda t: (0, 0)),
            pl.BlockSpec((_K // 2, _B * 128), lambda t: (0, 0)),
        ],
        out_specs=pl.BlockSpec((1, _B, _IB), lambda t: (t, 0, 0)),
        out_shape=jax.ShapeDtypeStruct((_NB, _B, _IB), jnp.float32),
        scratch_shapes=([pltpu.VMEM((_IB, 128), jnp.float32)] * 4
                        + [pltpu.SemaphoreType.DMA] * 4),
    )(g, labels3, nd, a_cm, e_tab)


def kernel(x, neighbour_distance, enc_W, enc_b, dec_W, bw_W, bw_b,
           neighbour_id, clustering_labels):
    dec_wt = dec_W.T

    enct, araw_t = _prep(x, enc_W, bw_W,
                         enc_b.reshape(_n, 1), bw_b.reshape(_n * _m, 1))
    a_cm = araw_t.reshape(_n, _m, _B).transpose(1, 2, 0).reshape(_m, _BJ)
    enc_r = enct.T.reshape(1, _BJ)
    labels3 = clustering_labels.reshape(_NB, 1, _IB)

    e_tab = _s_stage(labels3, neighbour_distance, a_cm, enc_r)

    nid2 = neighbour_id.reshape(_NW, _RPW)
    g3 = _sc_gather(dec_wt, nid2)
    g = g3.reshape(_N, _K // 2, 128)

    e_pair = e_tab.reshape(_K // 2, 2, _B, _n).transpose(0, 2, 1, 3) \
        .reshape(_K // 2, _B * 128)
    out6 = _main(g, labels3, neighbour_distance, a_cm, e_pair)
    return out6.transpose(1, 0, 2).reshape(_B, _N)


# speedup vs baseline: 4.4738x; 1.0136x over previous
import functools

import jax
import jax.numpy as jnp
from jax import lax
from jax.experimental import pallas as pl
from jax.experimental.pallas import tpu as pltpu
from jax.experimental.pallas import tpu_sc as plsc

_N = 10000
_n = 64
_m = 16
_K = 16
_B = 4
_MU = 600.0
_BJ = _B * _n

_IB = 2000
_NB = _N // _IB

_NW = 32
_RPW = _N * _K // _NW
_CH = 128
_NF = _RPW // _CH
_TAIL = _RPW - _NF * _CH
_NBUF = 4


def _prep_body(x_ref, encw_ref, bww_ref, encb_ref, bwb_ref,
               enct_out, araw_out):
    xt = jnp.transpose(x_ref[...])
    enct = lax.dot_general(encw_ref[...], xt,
                           (((1,), (0,)), ((), ())),
                           preferred_element_type=jnp.float32)
    enct = enct + encb_ref[...]
    logt = lax.dot_general(bww_ref[...], enct,
                           (((1,), (0,)), ((), ())),
                           preferred_element_type=jnp.float32)
    logt = logt + bwb_ref[...]
    tt = (60.0 / _MU) * (1.0 + jnp.exp(-logt))
    enct_out[...] = enct
    araw_out[...] = tt * tt


def _prep(x, enc_w, bw_w, encb_col, bwb_col):
    return pl.pallas_call(
        _prep_body,
        out_shape=(
            jax.ShapeDtypeStruct((_n, _B), jnp.float32),
            jax.ShapeDtypeStruct((_n * _m, _B), jnp.float32),
        ),
    )(x, enc_w, bw_w, encb_col, bwb_col)


def _window_block(lab_ref, nd_ref, acm_ref):
    lab = lab_ref[0]
    ioc = lax.broadcasted_iota(jnp.int32, (_m, _IB), 0)
    oht = jnp.where(lab == ioc, 1.0, 0.0)
    a_blk = lax.dot_general(oht, acm_ref[...],
                            (((0,), (0,)), ((), ())),
                            preferred_element_type=jnp.float32)
    ndb = nd_ref[...]
    return a_blk, ndb * ndb


def _s_body(lab_ref, nd_ref, acm_ref, encr_ref, e_out, s_acc):
    t = pl.program_id(0)
    a_blk, d2 = _window_block(lab_ref, nd_ref, acm_ref)
    ones_row = jnp.ones((1, _IB), jnp.float32)
    rows = []
    for k in range(_K):
        wk = jnp.maximum(1.0 - a_blk * d2[:, k:k + 1], 0.0)
        rows.append(lax.dot_general(ones_row, wk, (((1,), (0,)), ((), ())),
                                    preferred_element_type=jnp.float32))
    s_new = jnp.concatenate(rows, axis=0)

    @pl.when(t == 0)
    def _():
        s_acc[...] = s_new

    @pl.when(t != 0)
    def _():
        s_acc[...] = s_acc[...] + s_new

    @pl.when(t == _NB - 1)
    def _():
        e_out[...] = encr_ref[...] / s_acc[...]


def _s_stage(labels3, nd, a_cm, enc_r):
    return pl.pallas_call(
        _s_body,
        grid=(_NB,),
        in_specs=[
            pl.BlockSpec((1, 1, _IB), lambda t: (t, 0, 0)),
            pl.BlockSpec((_IB, _K), lambda t: (t, 0)),
            pl.BlockSpec((_m, _BJ), lambda t: (0, 0)),
            pl.BlockSpec((1, _BJ), lambda t: (0, 0)),
        ],
        out_specs=pl.BlockSpec((_K, _BJ), lambda t: (0, 0)),
        out_shape=jax.ShapeDtypeStruct((_K, _BJ), jnp.float32),
        scratch_shapes=[pltpu.VMEM((_K, _BJ), jnp.float32)],
    )(labels3, nd, a_cm, enc_r)


def _sc_gather_body(decwt_hbm, nid_hbm, out_hbm, idx_v, *scr):
    bufs = scr[:_NBUF]
    tailbuf = scr[_NBUF]
    gsems = scr[_NBUF + 1:2 * _NBUF + 1]
    ssems = scr[2 * _NBUF + 1:3 * _NBUF + 1]
    wid = lax.axis_index("s") * 2 + lax.axis_index("c")
    pltpu.sync_copy(nid_hbm.at[wid], idx_v)

    def gather(j):
        b = j % _NBUF
        return pltpu.async_copy(
            decwt_hbm.at[idx_v.at[pl.ds(j * _CH, _CH)]], bufs[b], gsems[b])

    def store(j):
        b = j % _NBUF
        return pltpu.async_copy(
            bufs[b], out_hbm.at[wid, pl.ds(j * _CH, _CH)], ssems[b])

    gops = [None] * _NF
    sops = [None] * _NF
    for j in range(_NF):
        if j >= _NBUF:
            sops[j - _NBUF].wait()
        gops[j] = gather(j)
        if j >= 2:
            gops[j - 2].wait()
            sops[j - 2] = store(j - 2)
    for j in range(max(_NF - 2, 0), _NF):
        gops[j].wait()
        sops[j] = store(j)
    tg = pltpu.async_copy(
        decwt_hbm.at[idx_v.at[pl.ds(_NF * _CH, _TAIL)]], tailbuf, gsems[0])
    tg.wait()
    pltpu.sync_copy(tailbuf, out_hbm.at[wid, pl.ds(_NF * _CH, _TAIL)])
    for j in range(max(_NF - _NBUF, 0), _NF):
        sops[j].wait()


def _sc_gather(dec_wt, nid2):
    scratch = ([pltpu.VMEM((_RPW,), jnp.int32)]
               + [pltpu.VMEM((_CH, _n), jnp.float32) for _ in range(_NBUF)]
               + [pltpu.VMEM((_TAIL, _n), jnp.float32)]
               + [pltpu.SemaphoreType.DMA for _ in range(2 * _NBUF)])
    fn = functools.partial(
        pl.kernel,
        out_type=jax.ShapeDtypeStruct((_NW, _RPW, _n), jnp.float32),
        mesh=plsc.VectorSubcoreMesh(core_axis_name="c", subcore_axis_name="s"),
        scratch_types=scratch,
        compiler_params=pltpu.CompilerParams(use_tc_tiling_on_sc=False),
    )(_sc_gather_body)
    return fn(dec_wt, nid2)


def _main_body(g_hbm, lab_ref, nd_ref, acm_ref, e_ref, out_ref, *scr):
    t = pl.program_id(0)
    lab = lab_ref[0]
    ioc = lax.broadcasted_iota(jnp.int32, (_m, _IB), 0)
    oht = jnp.where(lab == ioc, 1.0, 0.0)
    acm = acm_ref[...]
    acm_dup = jnp.concatenate(
        [jnp.concatenate([acm[:, b * _n:(b + 1) * _n]] * 2, axis=1)
         for b in range(_B)], axis=1)
    a2 = lax.dot_general(oht, acm_dup, (((0,), (0,)), ((), ())),
                         preferred_element_type=jnp.float32)
    a_b = [a2[:, b * 128:(b + 1) * 128] for b in range(_B)]
    nd = nd_ref[...]
    ndsq = nd * nd
    e_full = e_ref[...]
    r_k = lax.broadcasted_iota(jnp.int32, (_K, 128), 0)
    c_half = lax.broadcasted_iota(jnp.int32, (_K, 128), 1) // _n

    nq = _K // 2
    gbufs, sems = scr[:4], scr[4:]
    copies = [None] * nq

    def start(qq):
        copies[qq] = pltpu.make_async_copy(
            g_hbm.at[pl.ds(t * _IB, _IB), qq, :], gbufs[qq % 4], sems[qq % 4])
        copies[qq].start()

    for qq in range(3):
        start(qq)
    p = [None] * _B
    for q in range(nq):
        if q + 3 < nq:
            start(q + 3)
        copies[q].wait()
        gq = gbufs[q % 4][...]
        sel = jnp.where(r_k == 2 * q + c_half, 1.0, 0.0)
        d2q = lax.dot_general(ndsq, sel, (((1,), (0,)), ((), ())),
                              preferred_element_type=jnp.float32)
        for b in range(_B):
            erow = jnp.concatenate(
                [e_full[2 * q:2 * q + 1, b * _n:(b + 1) * _n],
                 e_full[2 * q + 1:2 * q + 2, b * _n:(b + 1) * _n]],
                axis=1)
            w = jnp.maximum(1.0 - a_b[b] * d2q, 0.0)
            term = w * gq * erow
            p[b] = term if p[b] is None else p[b] + term
    pcat = jnp.concatenate(p, axis=1)
    rr = lax.broadcasted_iota(jnp.int32, (4 * 128, _B), 0) // 128
    cc = lax.broadcasted_iota(jnp.int32, (4 * 128, _B), 1)
    bd = jnp.where(rr == cc, 1.0, 0.0)
    res = lax.dot_general(pcat, bd, (((1,), (0,)), ((), ())),
                          preferred_element_type=jnp.float32)
    out_ref[0] = jnp.transpose(res)


def _main(g, labels3, nd, a_cm, e_tab):
    return pl.pallas_call(
        _main_body,
        grid=(_NB,),
        in_specs=[
            pl.BlockSpec(memory_space=pl.ANY),
            pl.BlockSpec((1, 1, _IB), lambda t: (t, 0, 0)),
            pl.BlockSpec((_IB, _K), lambda t: (t, 0)),
            pl.BlockSpec((_m, _BJ), lambda t: (0, 0)),
            pl.BlockSpec((_K, _BJ), lambda t: (0, 0)),
        ],
        out_specs=pl.BlockSpec((1, _B, _IB), lambda t: (t, 0, 0)),
        out_shape=jax.ShapeDtypeStruct((_NB, _B, _IB), jnp.float32),
        scratch_shapes=([pltpu.VMEM((_IB, 128), jnp.float32)] * 4
                        + [pltpu.SemaphoreType.DMA] * 4),
    )(g, labels3, nd, a_cm, e_tab)


def kernel(x, neighbour_distance, enc_W, enc_b, dec_W, bw_W, bw_b,
           neighbour_id, clustering_labels):
    dec_wt = dec_W.T

    enct, araw_t = _prep(x, enc_W, bw_W,
                         enc_b.reshape(_n, 1), bw_b.reshape(_n * _m, 1))
    a_cm = araw_t.reshape(_n, _m, _B).transpose(1, 2, 0).reshape(_m, _BJ)
    enc_r = enct.T.reshape(1, _BJ)
    labels3 = clustering_labels.reshape(_NB, 1, _IB)

    e_tab = _s_stage(labels3, neighbour_distance, a_cm, enc_r)

    nid2 = neighbour_id.reshape(_NW, _RPW)
    g3 = _sc_gather(dec_wt, nid2)
    g = g3.reshape(_N, _K // 2, 128)

    out6 = _main(g, labels3, neighbour_distance, a_cm, e_tab)
    return out6.transpose(1, 0, 2).reshape(_B, _N)
